# baseline (speedup 1.0000x reference)
import jax
import jax.numpy as jnp
from jax import lax
from jax.experimental import pallas as pl
from jax.experimental.pallas import tpu as pltpu
from jax.experimental.pallas import tpu_sc as plsc

N = 10000
E = 320000
D = 128
HOPS = 5
EPS = 1e-5

NP = 10112
NPT = 632
C = 64
NTILES = 16
NW = 32
NCHUNK = 164
HALVES = ((0, 80), (80, 84))
EPT = NCHUNK * C
EP = EPT * NW
PAD_SRC = NP - 2
PAD_DST = NP - 1


def _mlp_body(x_ref, w1_ref, b1_ref, w2_ref, b2_ref, w3_ref, b3_ref, out_ref):
    x = x_ref[...]
    h = jnp.maximum(jnp.dot(x, w1_ref[...], preferred_element_type=jnp.float32)
                    + b1_ref[...], 0.0)
    h = jnp.maximum(jnp.dot(h, w2_ref[...], preferred_element_type=jnp.float32)
                    + b2_ref[...], 0.0)
    raw = jnp.dot(h, w3_ref[...], preferred_element_type=jnp.float32) + b3_ref[...]
    m = jnp.max(raw, axis=-1, keepdims=True)
    e = jnp.exp(raw - m)
    out_ref[...] = e / jnp.sum(e, axis=-1, keepdims=True)


def _mlp_alphas(static, W1, b1, W2, b2, W3, b3):
    blk = 1000
    grid = N // blk
    return pl.pallas_call(
        _mlp_body,
        grid=(grid,),
        in_specs=[
            pl.BlockSpec((blk, 16), lambda i: (i, 0)),
            pl.BlockSpec((16, 32), lambda i: (0, 0)),
            pl.BlockSpec((1, 32), lambda i: (0, 0)),
            pl.BlockSpec((32, 32), lambda i: (0, 0)),
            pl.BlockSpec((1, 32), lambda i: (0, 0)),
            pl.BlockSpec((32, HOPS + 1), lambda i: (0, 0)),
            pl.BlockSpec((1, HOPS + 1), lambda i: (0, 0)),
        ],
        out_specs=pl.BlockSpec((blk, HOPS + 1), lambda i: (i, 0)),
        out_shape=jax.ShapeDtypeStruct((N, HOPS + 1), jnp.float32),
    )(static, W1.T, b1[None, :], W2.T, b2[None, :], W3.T, b3[None, :])


def _edge_pipeline(src, sidx, ridx, bufs, gsems, cnt, ashared):
    def g_issue(u, b):
        pltpu.async_copy(src.at[sidx.at[u]], bufs[b], gsems[b])

    def g_wait(u, b):
        pltpu.make_async_copy(src.at[sidx.at[u]], bufs[b], gsems[b]).wait()

    g_issue(0, 0)

    def body(i, carry):
        for j in range(2):
            u = 2 * i + j
            nj = (j + 1) % 2

            @pl.when(u + 1 < cnt)
            def _():
                g_issue(u + 1, nj)
            g_wait(u, j)
            pltpu.sync_copy(bufs[j], ashared.at[ridx.at[u]], add=True)
        return carry
    lax.fori_loop(0, cnt // 2, body, 0)


def _make_sc_hop_body(nsrc):
    def _sc_body(*refs):
        srcs = refs[:nsrc]
        (s_hbm, r_hbm, out0, out1,
         sidx, ridx, bufA, bufB,
         gsA, gsB, ashared) = refs[nsrc:]
        bufs = (bufA, bufB)
        gsems = (gsA, gsB)
        c = lax.axis_index("c")
        s = lax.axis_index("s")
        w = c * NTILES + s
        row0 = s * NPT

        def _zero_row(i, carry):
            for j in range(D // 16):
                bufA[i, pl.ds(j * 16, 16)] = jnp.zeros((16,), jnp.float32)
            return carry
        lax.fori_loop(0, C, _zero_row, 0)
        for jz in range(NPT // C):
            pltpu.sync_copy(bufA, ashared.at[pl.ds(row0 + jz * C, C)])
        rem = NPT - (NPT // C) * C
        if rem:
            pltpu.sync_copy(bufA.at[pl.ds(0, rem)],
                            ashared.at[pl.ds(row0 + (NPT // C) * C, rem)])
        plsc.subcore_barrier()

        for src in srcs:
            for off, cnt in HALVES:
                pltpu.sync_copy(s_hbm.at[w, pl.ds(off, cnt)],
                                sidx.at[pl.ds(0, cnt)])
                pltpu.sync_copy(r_hbm.at[w, pl.ds(off, cnt)],
                                ridx.at[pl.ds(0, cnt)])
                _edge_pipeline(src, sidx, ridx, bufs, gsems, cnt, ashared)
        plsc.subcore_barrier()

        nck = -(-NPT // C)
        for jc in range(nck):
            r = row0 + jc * C
            nrow = C if jc < nck - 1 else NPT - (nck - 1) * C
            bslice = bufA.at[pl.ds(0, nrow)] if nrow != C else bufA
            pltpu.sync_copy(ashared.at[pl.ds(r, nrow)], bslice)

            @pl.when(c == 0)
            def _():
                pltpu.sync_copy(bslice, out0.at[pl.ds(r, nrow)])

            @pl.when(c == 1)
            def _():
                pltpu.sync_copy(bslice, out1.at[pl.ds(r, nrow)])
    return _sc_body


def _sc_hop(srcs, send_r, recv_r):
    mesh = plsc.VectorSubcoreMesh(core_axis_name="c", subcore_axis_name="s",
                                  num_cores=2, num_subcores=NTILES)
    return pl.kernel(
        _make_sc_hop_body(len(srcs)),
        out_type=[jax.ShapeDtypeStruct((NP, D), jnp.float32)] * 2,
        mesh=mesh,
        scratch_types=[
            pltpu.VMEM((HALVES[1][1], C), jnp.int32),
            pltpu.VMEM((HALVES[1][1], C), jnp.int32),
            pltpu.VMEM((C, D), jnp.float32),
            pltpu.VMEM((C, D), jnp.float32),
            pltpu.SemaphoreType.DMA,
            pltpu.SemaphoreType.DMA,
            pltpu.VMEM_SHARED((NP, D), jnp.float32),
        ],
    )(*srcs, send_r, recv_r)


def _sc_hops(h_pad, send_r, recv_r):
    parts = []
    srcs = [h_pad]
    for _ in range(HOPS):
        p0, p1 = _sc_hop(srcs, send_r, recv_r)
        parts.extend([p0, p1])
        srcs = [p0, p1]
    return parts


def _finalize_body(a_ref, h0_ref, *refs):
    (p00, p01, p10, p11, p20, p21, p30, p31, p40, p41,
     hp_ref, w_ref, b_ref, m_ref, out_ref) = refs
    a = a_ref[...]
    routed = a[:, 0:1] * h0_ref[...]
    pairs = ((p00, p01), (p10, p11), (p20, p21), (p30, p31), (p40, p41))
    for k, (pa, pb) in enumerate(pairs):
        routed = routed + a[:, k + 1:k + 2] * (pa[...] + pb[...])
    ms = jnp.mean(routed * routed, axis=-1, keepdims=True)
    inv = lax.rsqrt(ms + EPS)
    out_ref[...] = m_ref[...] * (hp_ref[...] + routed * inv * w_ref[...]
                                 + b_ref[...])


def _finalize(alphas, H_runoff, parts, H_prev, rms_weight, rms_bias, maskf):
    blk = 1000
    grid = N // blk
    row_spec = pl.BlockSpec((blk, D), lambda i: (i, 0))
    return pl.pallas_call(
        _finalize_body,
        grid=(grid,),
        in_specs=[
            pl.BlockSpec((blk, HOPS + 1), lambda i: (i, 0)),
            row_spec,
            *([row_spec] * (2 * HOPS)),
            row_spec,
            pl.BlockSpec((1, D), lambda i: (0, 0)),
            pl.BlockSpec((1, D), lambda i: (0, 0)),
            pl.BlockSpec((blk, 1), lambda i: (i, 0)),
        ],
        out_specs=row_spec,
        out_shape=jax.ShapeDtypeStruct((N, D), jnp.float32),
    )(alphas, H_runoff, *parts, H_prev, rms_weight[None, :], rms_bias[None, :],
      maskf)


def kernel(static, H_runoff, H_prev, edges, node_mask,
           W1, b1, W2, b2, W3, b3, rms_weight, rms_bias):
    alphas = _mlp_alphas(static, W1, b1, W2, b2, W3, b3)

    h_pad = jnp.zeros((NP, D), jnp.float32).at[:N].set(H_runoff)
    senders = jnp.full((EP,), PAD_SRC, jnp.int32).at[:E].set(edges[0])
    receivers = jnp.full((EP,), PAD_DST, jnp.int32).at[:E].set(edges[1])
    send_r = senders.reshape(NW, NCHUNK, C)
    recv_r = receivers.reshape(NW, NCHUNK, C)
    parts = _sc_hops(h_pad, send_r, recv_r)

    maskf = node_mask.astype(jnp.float32)[:, None]
    H_out = _finalize(alphas, H_runoff, parts, H_prev,
                      rms_weight, rms_bias, maskf)
    return (H_out, alphas)


# speedup vs baseline: 4.4201x; 4.4201x over previous
import jax
import jax.numpy as jnp
from jax import lax
from jax.experimental import pallas as pl
from jax.experimental.pallas import tpu as pltpu
from jax.experimental.pallas import tpu_sc as plsc

N = 10000
E = 320000
D = 128
HOPS = 5
EPS = 1e-5

NP = 10112
NPT = 632
C = 128
NTILES = 16
NW = 32
NCHUNK = 79
EPT = NCHUNK * C
EP = EPT * NW
PAD_SRC = NP - 2
PAD_DST = NP - 1
ZR = 64


def _mlp_body(x_ref, w1_ref, b1_ref, w2_ref, b2_ref, w3_ref, b3_ref, out_ref):
    x = x_ref[...]
    h = jnp.maximum(jnp.dot(x, w1_ref[...], preferred_element_type=jnp.float32)
                    + b1_ref[...], 0.0)
    h = jnp.maximum(jnp.dot(h, w2_ref[...], preferred_element_type=jnp.float32)
                    + b2_ref[...], 0.0)
    raw = jnp.dot(h, w3_ref[...], preferred_element_type=jnp.float32) + b3_ref[...]
    m = jnp.max(raw, axis=-1, keepdims=True)
    e = jnp.exp(raw - m)
    out_ref[...] = e / jnp.sum(e, axis=-1, keepdims=True)


def _mlp_alphas(static, W1, b1, W2, b2, W3, b3):
    blk = 1000
    grid = N // blk
    return pl.pallas_call(
        _mlp_body,
        grid=(grid,),
        in_specs=[
            pl.BlockSpec((blk, 16), lambda i: (i, 0)),
            pl.BlockSpec((16, 32), lambda i: (0, 0)),
            pl.BlockSpec((1, 32), lambda i: (0, 0)),
            pl.BlockSpec((32, 32), lambda i: (0, 0)),
            pl.BlockSpec((1, 32), lambda i: (0, 0)),
            pl.BlockSpec((32, HOPS + 1), lambda i: (0, 0)),
            pl.BlockSpec((1, HOPS + 1), lambda i: (0, 0)),
        ],
        out_specs=pl.BlockSpec((blk, HOPS + 1), lambda i: (i, 0)),
        out_shape=jax.ShapeDtypeStruct((N, HOPS + 1), jnp.float32),
    )(static, W1.T, b1[None, :], W2.T, b2[None, :], W3.T, b3[None, :])


def _make_sc_hop_body(nsrc):
    def _sc_body(*refs):
        srcs = refs[:nsrc]
        (s_hbm, r_hbm, out0, out1,
         sidx, ridx, rows, zbuf, ashared) = refs[nsrc:]
        c = lax.axis_index("c")
        s = lax.axis_index("s")
        w = c * NTILES + s
        row0 = s * NPT

        pltpu.sync_copy(s_hbm.at[w], sidx)
        pltpu.sync_copy(r_hbm.at[w], ridx)

        def _zero_row(i, carry):
            for j in range(D // 16):
                zbuf[i, pl.ds(j * 16, 16)] = jnp.zeros((16,), jnp.float32)
            return carry
        lax.fori_loop(0, ZR, _zero_row, 0)

        full, rem = divmod(NPT, ZR)
        for jz in range(full):
            pltpu.sync_copy(zbuf, ashared.at[pl.ds(row0 + jz * ZR, ZR)])
        if rem:
            pltpu.sync_copy(zbuf.at[pl.ds(0, rem)],
                            ashared.at[pl.ds(row0 + full * ZR, rem)])
        plsc.subcore_barrier()

        def _edge(ci, carry):
            for src in srcs:
                pltpu.sync_copy(src.at[sidx.at[ci]], rows)
                pltpu.sync_copy(rows, ashared.at[ridx.at[ci]], add=True)
            return carry
        lax.fori_loop(0, NCHUNK, _edge, 0)
        plsc.subcore_barrier()

        nck = -(-NPT // C)
        for jc in range(nck):
            r = row0 + jc * C
            nrow = C if jc < nck - 1 else NPT - (nck - 1) * C
            bslice = rows.at[pl.ds(0, nrow)] if nrow != C else rows
            pltpu.sync_copy(ashared.at[pl.ds(r, nrow)], bslice)

            @pl.when(c == 0)
            def _():
                pltpu.sync_copy(bslice, out0.at[pl.ds(r, nrow)])

            @pl.when(c == 1)
            def _():
                pltpu.sync_copy(bslice, out1.at[pl.ds(r, nrow)])
    return _sc_body


def _sc_hop(srcs, send_r, recv_r):
    mesh = plsc.VectorSubcoreMesh(core_axis_name="c", subcore_axis_name="s",
                                  num_cores=2, num_subcores=NTILES)
    return pl.kernel(
        _make_sc_hop_body(len(srcs)),
        out_type=[jax.ShapeDtypeStruct((NP, D), jnp.float32)] * 2,
        mesh=mesh,
        scratch_types=[
            pltpu.VMEM((NCHUNK, C), jnp.int32),
            pltpu.VMEM((NCHUNK, C), jnp.int32),
            pltpu.VMEM((C, D), jnp.float32),
            pltpu.VMEM((ZR, D), jnp.float32),
            pltpu.VMEM_SHARED((NP, D), jnp.float32),
        ],
    )(*srcs, send_r, recv_r)


def _combine_body(a_ref, b_ref, out_ref):
    out_ref[...] = a_ref[...] + b_ref[...]


def _combine(p0, p1):
    blk = 632
    spec = pl.BlockSpec((blk, D), lambda i: (i, 0))
    return pl.pallas_call(
        _combine_body,
        grid=(NP // blk,),
        in_specs=[spec, spec],
        out_specs=spec,
        out_shape=jax.ShapeDtypeStruct((NP, D), jnp.float32),
    )(p0, p1)


def _sc_hops(h_pad, send_r, recv_r):
    hops = []
    srcs = [h_pad]
    for k in range(HOPS):
        p0, p1 = _sc_hop(srcs, send_r, recv_r)
        if k < HOPS - 1:
            h = _combine(p0, p1)
            hops.append(h)
            srcs = [h]
        else:
            hops.append((p0, p1))
    return hops


def _finalize_body(a_ref, h0_ref, h1_ref, h2_ref, h3_ref, h4_ref,
                   p50_ref, p51_ref, hp_ref, w_ref, b_ref, m_ref, out_ref):
    a = a_ref[...]
    routed = a[:, 0:1] * h0_ref[...]
    for k, hk in enumerate((h1_ref, h2_ref, h3_ref, h4_ref)):
        routed = routed + a[:, k + 1:k + 2] * hk[...]
    routed = routed + a[:, HOPS:HOPS + 1] * (p50_ref[...] + p51_ref[...])
    ms = jnp.mean(routed * routed, axis=-1, keepdims=True)
    inv = lax.rsqrt(ms + EPS)
    out_ref[...] = m_ref[...] * (hp_ref[...] + routed * inv * w_ref[...]
                                 + b_ref[...])


def _finalize(alphas, H_runoff, hops, H_prev, rms_weight, rms_bias, maskf):
    blk = 1000
    grid = N // blk
    row_spec = pl.BlockSpec((blk, D), lambda i: (i, 0))
    h1, h2, h3, h4, (p50, p51) = hops
    return pl.pallas_call(
        _finalize_body,
        grid=(grid,),
        in_specs=[
            pl.BlockSpec((blk, HOPS + 1), lambda i: (i, 0)),
            row_spec,
            *([row_spec] * (HOPS + 1)),
            row_spec,
            pl.BlockSpec((1, D), lambda i: (0, 0)),
            pl.BlockSpec((1, D), lambda i: (0, 0)),
            pl.BlockSpec((blk, 1), lambda i: (i, 0)),
        ],
        out_specs=row_spec,
        out_shape=jax.ShapeDtypeStruct((N, D), jnp.float32),
    )(alphas, H_runoff, h1, h2, h3, h4, p50, p51, H_prev,
      rms_weight[None, :], rms_bias[None, :], maskf)


def kernel(static, H_runoff, H_prev, edges, node_mask,
           W1, b1, W2, b2, W3, b3, rms_weight, rms_bias):
    alphas = _mlp_alphas(static, W1, b1, W2, b2, W3, b3)

    h_pad = jnp.zeros((NP, D), jnp.float32).at[:N].set(H_runoff)
    senders = jnp.full((EP,), PAD_SRC, jnp.int32).at[:E].set(edges[0])
    receivers = jnp.full((EP,), PAD_DST, jnp.int32).at[:E].set(edges[1])
    send_r = senders.reshape(NW, NCHUNK, C)
    recv_r = receivers.reshape(NW, NCHUNK, C)
    parts = _sc_hops(h_pad, send_r, recv_r)

    maskf = node_mask.astype(jnp.float32)[:, None]
    H_out = _finalize(alphas, H_runoff, parts, H_prev,
                      rms_weight, rms_bias, maskf)
    return (H_out, alphas)
